# baseline (speedup 1.0000x reference)
import functools

import jax
import jax.numpy as jnp
from jax import lax
from jax.experimental import pallas as pl
from jax.experimental.pallas import tpu as pltpu
from jax.experimental.pallas import tpu_sc as plsc

NUM_CORES = 2
NUM_SUBCORES = 16
LANES = 16
NW = NUM_CORES * NUM_SUBCORES

CHUNK = 128
STRIPE = 32768


PAIR = 2048


def _transpose_body(x_ref, o_ref):
    stripe = x_ref.shape[1]
    for k in range(stripe // PAIR):
        t = jnp.transpose(x_ref[:, k * PAIR:(k + 1) * PAIR], (1, 0))
        half = PAIR // 2
        o_ref[k * half:(k + 1) * half, :] = jnp.concatenate(
            [t[0:half], t[half:]], axis=1)


def _retile_table(table_t):
    vocab = table_t.shape[1]
    nstripes = pl.cdiv(vocab, STRIPE)
    return pl.pallas_call(
        _transpose_body,
        grid=(nstripes,),
        in_specs=[pl.BlockSpec((64, STRIPE), lambda j: (0, j))],
        out_specs=pl.BlockSpec((STRIPE // 2, 128), lambda j: (j, 0)),
        out_shape=jax.ShapeDtypeStruct((nstripes * (STRIPE // 2), 128),
                                       jnp.float32),
        compiler_params=pltpu.CompilerParams(
            dimension_semantics=("parallel",)),
    )(table_t)


def _offset_kernel(table2, phys, par):
    batch = phys.shape[0] // 3
    b_per_w = batch // NW
    chunks_per_w = b_per_w // CHUNK
    mesh = plsc.VectorSubcoreMesh(core_axis_name="c", subcore_axis_name="s")

    @functools.partial(
        pl.kernel,
        out_type=jax.ShapeDtypeStruct((batch // 2, 128), jnp.float32),
        mesh=mesh,
        scratch_types=[
            pltpu.VMEM((b_per_w,), jnp.int32),
            pltpu.VMEM((b_per_w,), jnp.int32),
            pltpu.VMEM((b_per_w,), jnp.int32),
            pltpu.VMEM((b_per_w,), jnp.int32),
            pltpu.VMEM((b_per_w,), jnp.int32),
            pltpu.VMEM((b_per_w,), jnp.int32),
            pltpu.VMEM((CHUNK, 128), jnp.float32),
            pltpu.VMEM((CHUNK, 128), jnp.float32),
            pltpu.VMEM((CHUNK, 128), jnp.float32),
            pltpu.VMEM((CHUNK, 128), jnp.float32),
            pltpu.VMEM((CHUNK, 128), jnp.float32),
            pltpu.VMEM((CHUNK, 128), jnp.float32),
            pltpu.VMEM((CHUNK // 2, 128), jnp.float32),
            pltpu.SemaphoreType.DMA,
            pltpu.SemaphoreType.DMA,
        ],
        compiler_params=pltpu.CompilerParams(use_tc_tiling_on_sc=True),
    )
    def k(table_hbm, phys_hbm, par_hbm, out_hbm,
          i1_v, i2_v, i4_v, p1_v, p2_v, p4_v,
          a0_v, b0_v, c0_v, a1_v, b1_v, c1_v, o_v, sem0, sem1):
        wid = lax.axis_index("s") * NUM_CORES + lax.axis_index("c")
        base = wid * b_per_w
        ibase = wid * (3 * b_per_w)
        pltpu.sync_copy(phys_hbm.at[pl.ds(ibase, b_per_w)], i1_v)
        pltpu.sync_copy(phys_hbm.at[pl.ds(ibase + b_per_w, b_per_w)], i2_v)
        pltpu.sync_copy(phys_hbm.at[pl.ds(ibase + 2 * b_per_w, b_per_w)], i4_v)
        pltpu.sync_copy(par_hbm.at[pl.ds(ibase, b_per_w)], p1_v)
        pltpu.sync_copy(par_hbm.at[pl.ds(ibase + b_per_w, b_per_w)], p2_v)
        pltpu.sync_copy(par_hbm.at[pl.ds(ibase + 2 * b_per_w, b_per_w)], p4_v)

        bufs = [(a0_v, b0_v, c0_v, sem0), (a1_v, b1_v, c1_v, sem1)]

        def issue(g):
            a_v, b_v, c_v, sem = bufs[g % 2]
            off = g * CHUNK
            return (
                pltpu.async_copy(
                    table_hbm.at[i1_v.at[pl.ds(off, CHUNK)]], a_v, sem),
                pltpu.async_copy(
                    table_hbm.at[i2_v.at[pl.ds(off, CHUNK)]], b_v, sem),
                pltpu.async_copy(
                    table_hbm.at[i4_v.at[pl.ds(off, CHUNK)]], c_v, sem),
            )

        pending = issue(0)
        for g in range(chunks_per_w):
            nxt = issue(g + 1) if g + 1 < chunks_per_w else None
            for h in pending:
                h.wait()
            a_v, b_v, c_v, _ = bufs[g % 2]
            off = g * CHUNK

            @pl.loop(0, CHUNK // LANES)
            def _(rg, off=off, a_v=a_v, b_v=b_v, c_v=c_v):
                rbase = rg * LANES
                p1 = p1_v[pl.ds(off + rbase, LANES)]
                p2 = p2_v[pl.ds(off + rbase, LANES)]
                p4 = p4_v[pl.ds(off + rbase, LANES)]
                for j in range(LANES):
                    r = rbase + j
                    o1 = p1[j]
                    o2 = p2[j]
                    o4 = p4[j]
                    r2 = rg * (LANES // 2) + j // 2
                    rl = (j % 2) * 64
                    for c in range(0, 64, LANES):
                        o_v[r2, pl.ds(rl + c, LANES)] = (
                            a_v[r, pl.ds(o1 + c, LANES)]
                            - b_v[r, pl.ds(o2 + c, LANES)]
                            + c_v[r, pl.ds(o4 + c, LANES)]
                        )

            obase = pl.multiple_of((base + off) // 2, CHUNK // 2)
            pltpu.sync_copy(o_v, out_hbm.at[pl.ds(obase, CHUNK // 2)])
            pending = nxt

    return k(table2, phys, par)


def kernel(inputs, table):
    e1 = inputs[:, 0]
    e2 = inputs[:, 1]
    e3 = inputs[:, 2]
    e4 = inputs[:, 3]
    batch = inputs.shape[0]
    idx3 = jnp.stack([e1, e2, e4], axis=0)
    idx3 = idx3.reshape(3, NW, -1).transpose(1, 0, 2).reshape(-1)
    half = PAIR // 2
    phys = (idx3 // PAIR) * half + (idx3 % half)
    par = ((idx3 // half) % 2) * 64
    table2 = _retile_table(table.T)
    packed = _offset_kernel(table2, phys, par)
    offset_trick = packed.reshape(batch, 64)
    filters = jnp.concatenate(
        [e1.reshape(-1, 1), e2.reshape(-1, 1), e4.reshape(-1, 1)], axis=1)
    return (e1, e2, e3, e4, offset_trick, filters)
